# baseline (speedup 1.0000x reference)
import jax
import jax.numpy as jnp
from jax.experimental import pallas as pl
from jax.experimental.pallas import tpu as pltpu

_BLK = 1024
_NBUF = 6


def _in_copy(emb_ref, buf, in_sems, block, slot):
    return pltpu.make_async_copy(
        emb_ref.at[pl.ds(block * _BLK, _BLK), :], buf.at[slot], in_sems.at[slot]
    )


def _out_copy(out_ref, buf, out_sems, block, slot, b):
    return pltpu.make_async_copy(
        buf.at[slot], out_ref.at[b, pl.ds(block * _BLK, _BLK), :], out_sems.at[slot, b]
    )


def _bcast_kernel(emb_ref, out_ref, buf, in_sems, out_sems):
    i = pl.program_id(0)
    n = pl.num_programs(0)
    B = out_ref.shape[0]
    slot = jax.lax.rem(i, _NBUF)

    @pl.when(i == 0)
    def _():
        _in_copy(emb_ref, buf, in_sems, 0, 0).start()
        _in_copy(emb_ref, buf, in_sems, 1, 1).start()

    _in_copy(emb_ref, buf, in_sems, i, slot).wait()

    for b in range(B):
        _out_copy(out_ref, buf, out_sems, i, slot, b).start()

    @pl.when(i + 2 < n)
    def _():
        nslot = jax.lax.rem(i + 2, _NBUF)

        @pl.when(i + 2 >= _NBUF)
        def _():
            for b in range(B):
                _out_copy(out_ref, buf, out_sems, i + 2 - _NBUF, nslot, b).wait()

        _in_copy(emb_ref, buf, in_sems, i + 2, nslot).start()

    @pl.when(i == n - 1)
    def _():
        for k in range(_NBUF):
            blk = i - k
            for b in range(B):
                _out_copy(out_ref, buf, out_sems, blk, jax.lax.rem(blk, _NBUF), b).wait()


def kernel(x, emb_weight):
    B, L, D = x.shape
    grid = (L // _BLK,)
    out = pl.pallas_call(
        _bcast_kernel,
        grid=grid,
        in_specs=[pl.BlockSpec(memory_space=pl.MemorySpace.ANY)],
        out_specs=pl.BlockSpec(memory_space=pl.MemorySpace.ANY),
        out_shape=jax.ShapeDtypeStruct((B, L, D), emb_weight.dtype),
        scratch_shapes=[
            pltpu.VMEM((_NBUF, _BLK, D), emb_weight.dtype),
            pltpu.SemaphoreType.DMA((_NBUF,)),
            pltpu.SemaphoreType.DMA((_NBUF, B)),
        ],
    )(emb_weight)
    return out


# speedup vs baseline: 1.0216x; 1.0216x over previous
import jax
import jax.numpy as jnp
from jax.experimental import pallas as pl
from jax.experimental.pallas import tpu as pltpu

_BLK = 2048
_NBUF = 4


def _in_copy(emb_ref, buf, in_sems, block, slot):
    return pltpu.make_async_copy(
        emb_ref.at[pl.ds(block * _BLK, _BLK), :], buf.at[slot], in_sems.at[slot]
    )


def _out_copy(out_ref, buf, out_sems, block, slot, b):
    return pltpu.make_async_copy(
        buf.at[slot], out_ref.at[b, pl.ds(block * _BLK, _BLK), :], out_sems.at[slot, b]
    )


def _bcast_kernel(emb_ref, out_ref, buf, in_sems, out_sems):
    i = pl.program_id(0)
    n = pl.num_programs(0)
    B = out_ref.shape[0]
    slot = jax.lax.rem(i, _NBUF)

    @pl.when(i == 0)
    def _():
        _in_copy(emb_ref, buf, in_sems, 0, 0).start()
        _in_copy(emb_ref, buf, in_sems, 1, 1).start()

    _in_copy(emb_ref, buf, in_sems, i, slot).wait()

    for b in range(B):
        _out_copy(out_ref, buf, out_sems, i, slot, b).start()

    @pl.when(i + 2 < n)
    def _():
        nslot = jax.lax.rem(i + 2, _NBUF)

        @pl.when(i + 2 >= _NBUF)
        def _():
            for b in range(B):
                _out_copy(out_ref, buf, out_sems, i + 2 - _NBUF, nslot, b).wait()

        _in_copy(emb_ref, buf, in_sems, i + 2, nslot).start()

    @pl.when(i == n - 1)
    def _():
        for k in range(_NBUF):
            blk = i - k
            for b in range(B):
                _out_copy(out_ref, buf, out_sems, blk, jax.lax.rem(blk, _NBUF), b).wait()


def kernel(x, emb_weight):
    B, L, D = x.shape
    grid = (L // _BLK,)
    out = pl.pallas_call(
        _bcast_kernel,
        grid=grid,
        in_specs=[pl.BlockSpec(memory_space=pl.MemorySpace.ANY)],
        out_specs=pl.BlockSpec(memory_space=pl.MemorySpace.ANY),
        out_shape=jax.ShapeDtypeStruct((B, L, D), emb_weight.dtype),
        scratch_shapes=[
            pltpu.VMEM((_NBUF, _BLK, D), emb_weight.dtype),
            pltpu.SemaphoreType.DMA((_NBUF,)),
            pltpu.SemaphoreType.DMA((_NBUF, B)),
        ],
    )(emb_weight)
    return out
